# baseline (speedup 1.0000x reference)
import numpy as np

import jax
import jax.numpy as jnp
from jax import lax
from jax.experimental import pallas as pl
from jax.experimental.pallas import tpu as pltpu
from jax.experimental.pallas import tpu_sc as plsc

N = 10000
E = 320000
F = 128
K = 64
NB = 5
NMOL = 500
CUTOFF = 8.0

NPAD = 10112
TN = 632
GN = NPAD // TN
NW = 32
EW = E // NW
CW = 128
EC = E // CW
CHW = -(-EC // NW)
ZR = NPAD // 16
TE = 2000
GE = E // TE
LOG2 = 0.6931471805599453

_mesh = plsc.VectorSubcoreMesh(core_axis_name="c", subcore_axis_name="s")

_CP = np.empty((F,), np.int32)
for _q in range(F // 32):
    _CP[32 * _q: 32 * _q + 32: 2] = 32 * _q + np.arange(16)
    _CP[32 * _q + 1: 32 * _q + 32: 2] = 32 * _q + 16 + np.arange(16)
COLPERM = _CP


def _ssp(v):
    return jnp.maximum(v, 0.0) + jnp.log1p(jnp.exp(-jnp.abs(v))) - LOG2


def _d2_body(R512_hbm, ii_hbm, jj_hbm, dif_hbm,
             iirow0, iirow1, jjrow0, jjrow1, bufi0, bufi1, bufj0, bufj1,
             semi0, semi1, semj0, semj1, semw0, semw1):
    c = lax.axis_index("c")
    s = lax.axis_index("s")
    w = s * 2 + c
    iirow = (iirow0, iirow1)
    jjrow = (jjrow0, jjrow1)
    bufi = (bufi0, bufi1)
    bufj = (bufj0, bufj1)
    semi = (semi0, semi1)
    semj = (semj0, semj1)
    semw = (semw0, semw1)

    pltpu.sync_copy(ii_hbm.at[w], iirow[0])
    pltpu.sync_copy(jj_hbm.at[w], jjrow[0])
    pltpu.async_copy(R512_hbm.at[iirow[0]], bufi[0], semi[0])
    pltpu.async_copy(R512_hbm.at[jjrow[0]], bufj[0], semj[0])

    def outer(t2, carry):
        for ph in (0, 1):
            tt = t2 * 2 + ph
            X, Nx = ph, 1 - ph
            cid = w + tt * NW
            cidn = w + (tt + 1) * NW

            @pl.when(cidn < EC)
            def _():
                @pl.when(tt >= 1)
                def _():
                    pltpu.make_async_copy(
                        bufi[Nx], dif_hbm.at[pl.ds((cid - NW) * CW, CW)], semw[Nx]
                    ).wait()

                pltpu.sync_copy(ii_hbm.at[cidn], iirow[Nx])
                pltpu.sync_copy(jj_hbm.at[cidn], jjrow[Nx])
                pltpu.async_copy(R512_hbm.at[iirow[Nx]], bufi[Nx], semi[Nx])
                pltpu.async_copy(R512_hbm.at[jjrow[Nx]], bufj[Nx], semj[Nx])

            @pl.when(cid < EC)
            def _():
                pltpu.make_async_copy(R512_hbm.at[iirow[X]], bufi[X], semi[X]).wait()
                pltpu.make_async_copy(R512_hbm.at[jjrow[X]], bufj[X], semj[X]).wait()

                def sub(r, carry2):
                    sl = pl.ds(0, 16)
                    bufi[X][r, sl] = bufi[X][r, sl] - bufj[X][r, sl]
                    return carry2

                lax.fori_loop(0, CW, sub, 0)
                pltpu.async_copy(bufi[X], dif_hbm.at[pl.ds(cid * CW, CW)], semw[X])

        return carry

    lax.fori_loop(0, (CHW + 1) // 2, outer, 0)
    for X in (0, 1):
        pltpu.make_async_copy(bufi[X], dif_hbm.at[pl.ds(0, CW)], semw[X]).wait()


_d2_call = pl.kernel(
    _d2_body,
    out_type=jax.ShapeDtypeStruct((E, F), jnp.float32),
    mesh=_mesh,
    scratch_types=[
        pltpu.VMEM((CW,), jnp.int32),
        pltpu.VMEM((CW,), jnp.int32),
        pltpu.VMEM((CW,), jnp.int32),
        pltpu.VMEM((CW,), jnp.int32),
        pltpu.VMEM((CW, F), jnp.float32),
        pltpu.VMEM((CW, F), jnp.float32),
        pltpu.VMEM((CW, F), jnp.float32),
        pltpu.VMEM((CW, F), jnp.float32),
        pltpu.SemaphoreType.DMA,
        pltpu.SemaphoreType.DMA,
        pltpu.SemaphoreType.DMA,
        pltpu.SemaphoreType.DMA,
        pltpu.SemaphoreType.DMA,
        pltpu.SemaphoreType.DMA,
    ],
)


def _make_edge_call(b):
    def body(G_hbm, h_hbm, ij_hbm, P_hbm,
             ij0, ij1, hbuf0, hbuf1, gbuf, agg,
             semg0, semg1, semgl, sems0, sems1):
        c = lax.axis_index("c")
        s = lax.axis_index("s")
        w = s * 2 + c
        ijb = (ij0, ij1)
        hbuf = (hbuf0, hbuf1)
        semg = (semg0, semg1)
        sems = (sems0, sems1)

        def zb(r, carry):
            for q in range(8):
                hbuf0[r, pl.ds(q * 16, 16)] = jnp.zeros((16,), jnp.float32)
            return carry

        lax.fori_loop(0, CW, zb, 0)
        for t in range(4):
            pltpu.sync_copy(hbuf0, agg.at[pl.ds(s * ZR + t * CW, CW)])
        pltpu.sync_copy(hbuf0.at[pl.ds(0, ZR - 4 * CW)],
                        agg.at[pl.ds(s * ZR + 4 * CW, ZR - 4 * CW)])
        plsc.subcore_barrier()

        pltpu.sync_copy(ij_hbm.at[w], ijb[0])
        pltpu.async_copy(h_hbm.at[ijb[0].at[1]], hbuf[0], semg[0])
        pltpu.async_copy(G_hbm.at[pl.ds(w * CW, CW)], gbuf, semgl)

        def outer(t2, carry):
            for ph in (0, 1):
                tt = t2 * 2 + ph
                X, Nx = ph, 1 - ph
                cid = w + tt * NW
                cidn = w + (tt + 1) * NW

                @pl.when(cidn < EC)
                def _():
                    @pl.when(tt >= 1)
                    def _():
                        pltpu.make_async_copy(
                            hbuf[Nx], agg.at[ijb[Nx].at[0]], sems[Nx]
                        ).wait()

                    pltpu.sync_copy(ij_hbm.at[cidn], ijb[Nx])
                    pltpu.async_copy(h_hbm.at[ijb[Nx].at[1]], hbuf[Nx], semg[Nx])

                @pl.when(cid < EC)
                def _():
                    pltpu.make_async_copy(h_hbm.at[ijb[X].at[1]], hbuf[X], semg[X]).wait()
                    pltpu.make_async_copy(
                        G_hbm.at[pl.ds(cid * CW, CW)], gbuf, semgl
                    ).wait()

                    def mul(r, carry2):
                        for q in range(8):
                            sl = pl.ds(q * 16, 16)
                            hbuf[X][r, sl] = hbuf[X][r, sl] * gbuf[r, sl]
                        return carry2

                    lax.fori_loop(0, CW, mul, 0)

                    @pl.when(cidn < EC)
                    def _():
                        pltpu.async_copy(
                            G_hbm.at[pl.ds(cidn * CW, CW)], gbuf, semgl
                        )

                    pltpu.async_copy(hbuf[X], agg.at[ijb[X].at[0]], sems[X], add=True)

            return carry

        lax.fori_loop(0, (CHW + 1) // 2, outer, 0)
        for X in (0, 1):
            pltpu.make_async_copy(hbuf[X], agg.at[ijb[X].at[0]], sems[X]).wait()
        plsc.subcore_barrier()
        pltpu.sync_copy(agg.at[pl.ds(s * ZR, ZR)], P_hbm.at[c, pl.ds(s * ZR, ZR)])

    return pl.kernel(
        body,
        out_type=jax.ShapeDtypeStruct((2, NPAD, F), jnp.float32),
        mesh=_mesh,
        scratch_types=[
            pltpu.VMEM((2, CW), jnp.int32),
            pltpu.VMEM((2, CW), jnp.int32),
            pltpu.VMEM((CW, F), jnp.float32),
            pltpu.VMEM((CW, F), jnp.float32),
            pltpu.VMEM((CW, F), jnp.float32),
            pltpu.VMEM_SHARED((NPAD, F), jnp.float32),
            pltpu.SemaphoreType.DMA,
            pltpu.SemaphoreType.DMA,
            pltpu.SemaphoreType.DMA,
            pltpu.SemaphoreType.DMA,
            pltpu.SemaphoreType.DMA,
        ],
    )


_edge_calls = [_make_edge_call(b) for b in range(NB)]


def _init_body(Z_ref, M_ref, A_ref, B_ref, emb_ref, wm_ref, wa_ref, wb_ref,
               Wm0_ref, x_ref, h_ref):
    z = Z_ref[...]
    onehot = (lax.broadcasted_iota(jnp.int32, (TN, 96), 1) == z).astype(jnp.float32)
    x = jnp.dot(onehot, emb_ref[...], preferred_element_type=jnp.float32)
    x = x + M_ref[...] * wm_ref[...] + A_ref[...] * wa_ref[...] + B_ref[...] * wb_ref[...]
    x_ref[...] = x
    h_ref[...] = _ssp(jnp.dot(x, Wm0_ref[...], preferred_element_type=jnp.float32))


def _init_call(Zp, Mp, Ap, Bp, emb96, wm2, wa2, wb2, Wm0):
    return pl.pallas_call(
        _init_body,
        grid=(GN,),
        in_specs=[
            pl.BlockSpec((TN, 1), lambda i: (i, 0)),
            pl.BlockSpec((TN, 1), lambda i: (i, 0)),
            pl.BlockSpec((TN, 1), lambda i: (i, 0)),
            pl.BlockSpec((TN, 1), lambda i: (i, 0)),
            pl.BlockSpec((96, F), lambda i: (0, 0)),
            pl.BlockSpec((1, F), lambda i: (0, 0)),
            pl.BlockSpec((1, F), lambda i: (0, 0)),
            pl.BlockSpec((1, F), lambda i: (0, 0)),
            pl.BlockSpec((F, F), lambda i: (0, 0)),
        ],
        out_specs=[
            pl.BlockSpec((TN, F), lambda i: (i, 0)),
            pl.BlockSpec((TN, F), lambda i: (i, 0)),
        ],
        out_shape=[
            jax.ShapeDtypeStruct((NPAD, F), jnp.float32),
            jax.ShapeDtypeStruct((NPAD, F), jnp.float32),
        ],
    )(Zp, Mp, Ap, Bp, emb96, wm2, wa2, wb2, Wm0)


def _g_body(dif_ref, wr_ref, g_ref):
    d = dif_ref[...]
    d2 = jnp.sum(d * d, axis=1, keepdims=True)
    dij = jnp.sqrt(d2 + 1e-12)
    mu = lax.broadcasted_iota(jnp.int32, (1, K), 1).astype(jnp.float32) * (CUTOFF / (K - 1))
    t = dij - mu
    rbf = jnp.exp(-4.0 * t * t)
    fc = 0.5 * (jnp.cos(jnp.pi * jnp.clip(dij / CUTOFF, 0.0, 1.0)) + 1.0)
    rbf = rbf * fc
    g_ref[...] = jnp.dot(rbf, wr_ref[...], preferred_element_type=jnp.float32)


def _g_call(d2c, W_rbf_b):
    return pl.pallas_call(
        _g_body,
        grid=(GE,),
        in_specs=[
            pl.BlockSpec((TE, F), lambda i: (i, 0)),
            pl.BlockSpec((K, F), lambda i: (0, 0)),
        ],
        out_specs=pl.BlockSpec((TE, F), lambda i: (i, 0)),
        out_shape=jax.ShapeDtypeStruct((E, F), jnp.float32),
    )(d2c, W_rbf_b)


def _upd_body(x_ref, P_ref, Wu_ref, bu_ref, Wm_ref, xo_ref, ho_ref):
    agg = P_ref[0] + P_ref[1]
    u = _ssp(jnp.dot(agg, Wu_ref[...], preferred_element_type=jnp.float32) + bu_ref[...])
    xn = x_ref[...] + u
    xo_ref[...] = xn
    ho_ref[...] = _ssp(jnp.dot(xn, Wm_ref[...], preferred_element_type=jnp.float32))


def _upd_call(x, P, Wu, bu, Wm):
    return pl.pallas_call(
        _upd_body,
        grid=(GN,),
        in_specs=[
            pl.BlockSpec((TN, F), lambda i: (i, 0)),
            pl.BlockSpec((2, TN, F), lambda i: (0, i, 0)),
            pl.BlockSpec((F, F), lambda i: (0, 0)),
            pl.BlockSpec((1, F), lambda i: (0, 0)),
            pl.BlockSpec((F, F), lambda i: (0, 0)),
        ],
        out_specs=[
            pl.BlockSpec((TN, F), lambda i: (i, 0)),
            pl.BlockSpec((TN, F), lambda i: (i, 0)),
        ],
        out_shape=[
            jax.ShapeDtypeStruct((NPAD, F), jnp.float32),
            jax.ShapeDtypeStruct((NPAD, F), jnp.float32),
        ],
    )(x, P, Wu, bu, Wm)


def _fin_body(x_ref, P_ref, Wu_ref, bu_ref, Wo_ref, bo_ref, mol_ref,
              vec_ref, acc_ref):
    i = pl.program_id(0)
    agg = P_ref[0] + P_ref[1]
    xn = x_ref[...] + _ssp(jnp.dot(agg, Wu_ref[...], preferred_element_type=jnp.float32) + bu_ref[...])
    out = jnp.dot(xn, Wo_ref[...], preferred_element_type=jnp.float32) + bo_ref[...]
    col = lax.broadcasted_iota(jnp.int32, (TN, F), 1)
    vec = jnp.where(col == 0, out, jnp.where(col == 1, jnp.maximum(out, 0.0), 0.0))
    vec_ref[...] = vec
    onehot = (lax.broadcasted_iota(jnp.int32, (NMOL, TN), 0) == mol_ref[0]).astype(jnp.float32)

    @pl.when(i == 0)
    def _():
        acc_ref[...] = jnp.zeros((NMOL, F), jnp.float32)

    acc_ref[...] += jnp.dot(onehot, vec, preferred_element_type=jnp.float32)


def _fin_call(x, P, Wu, bu, Wo, bo, mol2):
    return pl.pallas_call(
        _fin_body,
        grid=(GN,),
        in_specs=[
            pl.BlockSpec((TN, F), lambda i: (i, 0)),
            pl.BlockSpec((2, TN, F), lambda i: (0, i, 0)),
            pl.BlockSpec((F, F), lambda i: (0, 0)),
            pl.BlockSpec((1, F), lambda i: (0, 0)),
            pl.BlockSpec((F, F), lambda i: (0, 0)),
            pl.BlockSpec((1, F), lambda i: (0, 0)),
            pl.BlockSpec((1, 1, TN), lambda i: (i, 0, 0)),
        ],
        out_specs=[
            pl.BlockSpec((TN, F), lambda i: (i, 0)),
            pl.BlockSpec((NMOL, F), lambda i: (0, 0)),
        ],
        out_shape=[
            jax.ShapeDtypeStruct((NPAD, F), jnp.float32),
            jax.ShapeDtypeStruct((NMOL, F), jnp.float32),
        ],
    )(x, P, Wu, bu, Wo, bo, mol2)


def kernel(Z, R, M, QaAlpha, QaBeta, idx_i, idx_j, mol_idx,
           embed, w_m, w_a, w_b, W_rbf, W_msg, W_upd, b_upd, W_out, b_out):
    f32 = jnp.float32
    ii = idx_i.astype(jnp.int32)
    jj = idx_j.astype(jnp.int32)
    ii3 = ii.reshape(EC, CW)
    jj3 = jj.reshape(EC, CW)
    ij3 = jnp.stack([ii3, jj3], axis=1)

    W_rbf_p = W_rbf[:, :, COLPERM]
    W_msg_p = W_msg[:, :, COLPERM]
    W_upd_p = W_upd[:, COLPERM, :]

    R512 = jnp.zeros((N, F), f32).at[:, :3].set(R.astype(f32))
    dif = _d2_call(R512, ii3, jj3)

    Zp = jnp.zeros((NPAD, 1), jnp.int32).at[:N, 0].set(Z.astype(jnp.int32))
    Mp = jnp.zeros((NPAD, 1), f32).at[:N, 0].set(M)
    Ap = jnp.zeros((NPAD, 1), f32).at[:N, 0].set(QaAlpha)
    Bp = jnp.zeros((NPAD, 1), f32).at[:N, 0].set(QaBeta)
    emb96 = jnp.zeros((96, F), f32).at[:95].set(embed)
    wm2 = w_m.reshape(1, F)
    wa2 = w_a.reshape(1, F)
    wb2 = w_b.reshape(1, F)

    x, h = _init_call(Zp, Mp, Ap, Bp, emb96, wm2, wa2, wb2, W_msg_p[0])

    vec = acc = None
    for b in range(NB):
        G = _g_call(dif, W_rbf_p[b])
        P = _edge_calls[b](G, h, ij3)
        if b < NB - 1:
            x, h = _upd_call(x, P, W_upd_p[b], b_upd[b].reshape(1, F), W_msg_p[b + 1])
        else:
            Wo = jnp.zeros((F, F), f32).at[:, :2].set(W_out)
            bo = jnp.zeros((1, F), f32).at[0, :2].set(b_out)
            molp = (jnp.full((NPAD,), 1000, jnp.int32)
                    .at[:N].set(mol_idx.astype(jnp.int32)).reshape(GN, 1, TN))
            vec, acc = _fin_call(x, P, W_upd_p[b], b_upd[b].reshape(1, F), Wo, bo, molp)

    charges = acc[:, 0]
    Qa = vec[:N, 0]
    I_mol = acc[:, 1]
    return (charges, Qa, I_mol)


# speedup vs baseline: 1.7782x; 1.7782x over previous
import numpy as np

import jax
import jax.numpy as jnp
from jax import lax
from jax.experimental import pallas as pl
from jax.experimental.pallas import tpu as pltpu
from jax.experimental.pallas import tpu_sc as plsc

N = 10000
E = 320000
F = 128
K = 64
NB = 5
NMOL = 500
CUTOFF = 8.0

NPAD = 10112
TN = 632
GN = NPAD // TN
NW = 32
EW = E // NW
CW = 128
EC = E // CW
CHW = -(-EC // NW)
ZR = NPAD // 16
TE = 2000
GE = E // TE
LOG2 = 0.6931471805599453

_mesh = plsc.VectorSubcoreMesh(core_axis_name="c", subcore_axis_name="s")

_CP = np.empty((F,), np.int32)
for _q in range(F // 32):
    _CP[32 * _q: 32 * _q + 32: 2] = 32 * _q + np.arange(16)
    _CP[32 * _q + 1: 32 * _q + 32: 2] = 32 * _q + 16 + np.arange(16)
COLPERM = _CP


def _ssp(v):
    return jnp.maximum(v, 0.0) + jnp.log1p(jnp.exp(-jnp.abs(v))) - LOG2


def _d2_body(R512_hbm, ii_hbm, jj_hbm, dif_hbm,
             iirow0, iirow1, jjrow0, jjrow1, bufi0, bufi1, bufj0, bufj1,
             semi0, semi1, semj0, semj1, semw0, semw1):
    c = lax.axis_index("c")
    s = lax.axis_index("s")
    w = s * 2 + c
    iirow = (iirow0, iirow1)
    jjrow = (jjrow0, jjrow1)
    bufi = (bufi0, bufi1)
    bufj = (bufj0, bufj1)
    semi = (semi0, semi1)
    semj = (semj0, semj1)
    semw = (semw0, semw1)

    pltpu.sync_copy(ii_hbm.at[w], iirow[0])
    pltpu.sync_copy(jj_hbm.at[w], jjrow[0])
    pltpu.async_copy(R512_hbm.at[iirow[0]], bufi[0], semi[0])
    pltpu.async_copy(R512_hbm.at[jjrow[0]], bufj[0], semj[0])

    def outer(t2, carry):
        for ph in (0, 1):
            tt = t2 * 2 + ph
            X, Nx = ph, 1 - ph
            cid = w + tt * NW
            cidn = w + (tt + 1) * NW

            @pl.when(cidn < EC)
            def _():
                @pl.when(tt >= 1)
                def _():
                    pltpu.make_async_copy(
                        bufi[Nx], dif_hbm.at[pl.ds((cid - NW) * CW, CW)], semw[Nx]
                    ).wait()

                pltpu.sync_copy(ii_hbm.at[cidn], iirow[Nx])
                pltpu.sync_copy(jj_hbm.at[cidn], jjrow[Nx])
                pltpu.async_copy(R512_hbm.at[iirow[Nx]], bufi[Nx], semi[Nx])
                pltpu.async_copy(R512_hbm.at[jjrow[Nx]], bufj[Nx], semj[Nx])

            @pl.when(cid < EC)
            def _():
                pltpu.make_async_copy(R512_hbm.at[iirow[X]], bufi[X], semi[X]).wait()
                pltpu.make_async_copy(R512_hbm.at[jjrow[X]], bufj[X], semj[X]).wait()

                def sub(r, carry2):
                    sl = pl.ds(0, 16)
                    bufi[X][r, sl] = bufi[X][r, sl] - bufj[X][r, sl]
                    return carry2

                lax.fori_loop(0, CW, sub, 0)
                pltpu.async_copy(bufi[X], dif_hbm.at[pl.ds(cid * CW, CW)], semw[X])

        return carry

    lax.fori_loop(0, (CHW + 1) // 2, outer, 0)
    for X in (0, 1):
        pltpu.make_async_copy(bufi[X], dif_hbm.at[pl.ds(0, CW)], semw[X]).wait()


_d2_call = pl.kernel(
    _d2_body,
    out_type=jax.ShapeDtypeStruct((E, F), jnp.float32),
    mesh=_mesh,
    scratch_types=[
        pltpu.VMEM((CW,), jnp.int32),
        pltpu.VMEM((CW,), jnp.int32),
        pltpu.VMEM((CW,), jnp.int32),
        pltpu.VMEM((CW,), jnp.int32),
        pltpu.VMEM((CW, F), jnp.float32),
        pltpu.VMEM((CW, F), jnp.float32),
        pltpu.VMEM((CW, F), jnp.float32),
        pltpu.VMEM((CW, F), jnp.float32),
        pltpu.SemaphoreType.DMA,
        pltpu.SemaphoreType.DMA,
        pltpu.SemaphoreType.DMA,
        pltpu.SemaphoreType.DMA,
        pltpu.SemaphoreType.DMA,
        pltpu.SemaphoreType.DMA,
    ],
)


def _make_edge_call(b):
    def body(G_hbm, h_hbm, ij_hbm, P_hbm,
             ij0, ij1, ij2, hbuf0, hbuf1, gbuf, agg,
             semg0, semg1, semgl, sems0, sems1, semij0, semij1, semij2):
        c = lax.axis_index("c")
        s = lax.axis_index("s")
        w = s * 2 + c
        ijb = (ij0, ij1, ij2)
        hbuf = (hbuf0, hbuf1)
        semg = (semg0, semg1)
        sems = (sems0, sems1)
        semij = (semij0, semij1, semij2)

        def zb(r, carry):
            for q in range(8):
                hbuf0[r, pl.ds(q * 16, 16)] = jnp.zeros((16,), jnp.float32)
            return carry

        lax.fori_loop(0, CW, zb, 0)
        for t in range(4):
            pltpu.sync_copy(hbuf0, agg.at[pl.ds(s * ZR + t * CW, CW)])
        pltpu.sync_copy(hbuf0.at[pl.ds(0, ZR - 4 * CW)],
                        agg.at[pl.ds(s * ZR + 4 * CW, ZR - 4 * CW)])
        plsc.subcore_barrier()

        pltpu.sync_copy(ij_hbm.at[w], ijb[0])
        pltpu.async_copy(h_hbm.at[ijb[0].at[1]], hbuf[0], semg[0])
        pltpu.async_copy(G_hbm.at[b, pl.ds(w * CW, CW)], gbuf, semgl)

        @pl.when((w + NW) < EC)
        def _():
            pltpu.async_copy(ij_hbm.at[w + NW], ijb[1], semij[1])

        def outer(t2, carry):
            for ph in range(6):
                tt = t2 * 6 + ph
                X, Nx = ph % 2, 1 - ph % 2
                J, Jn, Jp = ph % 3, (ph + 1) % 3, (ph + 2) % 3
                cid = w + tt * NW
                cidn = w + (tt + 1) * NW
                cid2 = w + (tt + 2) * NW

                @pl.when(cidn < EC)
                def _():
                    @pl.when(tt >= 1)
                    def _():
                        pltpu.make_async_copy(
                            hbuf[Nx], agg.at[ijb[Jn].at[0]], sems[Nx]
                        ).wait()

                    @pl.when(cid2 < EC)
                    def _():
                        pltpu.async_copy(ij_hbm.at[cid2], ijb[Jp], semij[Jp])

                    pltpu.make_async_copy(ij_hbm.at[cidn], ijb[Jn], semij[Jn]).wait()
                    pltpu.async_copy(h_hbm.at[ijb[Jn].at[1]], hbuf[Nx], semg[Nx])

                @pl.when(cid < EC)
                def _():
                    pltpu.make_async_copy(h_hbm.at[ijb[J].at[1]], hbuf[X], semg[X]).wait()
                    pltpu.make_async_copy(
                        G_hbm.at[b, pl.ds(cid * CW, CW)], gbuf, semgl
                    ).wait()

                    def mul(r, carry2):
                        for q in range(8):
                            sl = pl.ds(q * 16, 16)
                            hbuf[X][r, sl] = hbuf[X][r, sl] * gbuf[r, sl]
                        return carry2

                    lax.fori_loop(0, CW, mul, 0)

                    @pl.when(cidn < EC)
                    def _():
                        pltpu.async_copy(
                            G_hbm.at[b, pl.ds(cidn * CW, CW)], gbuf, semgl
                        )

                    pltpu.async_copy(hbuf[X], agg.at[ijb[J].at[0]], sems[X], add=True)

            return carry

        lax.fori_loop(0, (CHW + 5) // 6, outer, 0)
        @pl.when(w + 78 * NW < EC)
        def _():
            pltpu.make_async_copy(hbuf[0], agg.at[ijb[0].at[0]], sems[0]).wait()
            pltpu.make_async_copy(hbuf[1], agg.at[ijb[2].at[0]], sems[1]).wait()

        @pl.when(w + 78 * NW >= EC)
        def _():
            pltpu.make_async_copy(hbuf[1], agg.at[ijb[2].at[0]], sems[1]).wait()
            pltpu.make_async_copy(hbuf[0], agg.at[ijb[1].at[0]], sems[0]).wait()
        plsc.subcore_barrier()
        pltpu.sync_copy(agg.at[pl.ds(s * ZR, ZR)], P_hbm.at[c, pl.ds(s * ZR, ZR)])

    return pl.kernel(
        body,
        out_type=jax.ShapeDtypeStruct((2, NPAD, F), jnp.float32),
        mesh=_mesh,
        scratch_types=[
            pltpu.VMEM((2, CW), jnp.int32),
            pltpu.VMEM((2, CW), jnp.int32),
            pltpu.VMEM((2, CW), jnp.int32),
            pltpu.VMEM((CW, F), jnp.float32),
            pltpu.VMEM((CW, F), jnp.float32),
            pltpu.VMEM((CW, F), jnp.float32),
            pltpu.VMEM_SHARED((NPAD, F), jnp.float32),
            pltpu.SemaphoreType.DMA,
            pltpu.SemaphoreType.DMA,
            pltpu.SemaphoreType.DMA,
            pltpu.SemaphoreType.DMA,
            pltpu.SemaphoreType.DMA,
            pltpu.SemaphoreType.DMA,
            pltpu.SemaphoreType.DMA,
            pltpu.SemaphoreType.DMA,
        ],
    )


_edge_calls = [_make_edge_call(b) for b in range(NB)]


def _init_body(Z_ref, M_ref, A_ref, B_ref, emb_ref, wm_ref, wa_ref, wb_ref,
               Wm0_ref, x_ref, h_ref):
    z = Z_ref[...]
    onehot = (lax.broadcasted_iota(jnp.int32, (TN, 96), 1) == z).astype(jnp.float32)
    x = jnp.dot(onehot, emb_ref[...], preferred_element_type=jnp.float32)
    x = x + M_ref[...] * wm_ref[...] + A_ref[...] * wa_ref[...] + B_ref[...] * wb_ref[...]
    x_ref[...] = x
    h_ref[...] = _ssp(jnp.dot(x, Wm0_ref[...], preferred_element_type=jnp.float32))


def _init_call(Zp, Mp, Ap, Bp, emb96, wm2, wa2, wb2, Wm0):
    return pl.pallas_call(
        _init_body,
        grid=(GN,),
        in_specs=[
            pl.BlockSpec((TN, 1), lambda i: (i, 0)),
            pl.BlockSpec((TN, 1), lambda i: (i, 0)),
            pl.BlockSpec((TN, 1), lambda i: (i, 0)),
            pl.BlockSpec((TN, 1), lambda i: (i, 0)),
            pl.BlockSpec((96, F), lambda i: (0, 0)),
            pl.BlockSpec((1, F), lambda i: (0, 0)),
            pl.BlockSpec((1, F), lambda i: (0, 0)),
            pl.BlockSpec((1, F), lambda i: (0, 0)),
            pl.BlockSpec((F, F), lambda i: (0, 0)),
        ],
        out_specs=[
            pl.BlockSpec((TN, F), lambda i: (i, 0)),
            pl.BlockSpec((TN, F), lambda i: (i, 0)),
        ],
        out_shape=[
            jax.ShapeDtypeStruct((NPAD, F), jnp.float32),
            jax.ShapeDtypeStruct((NPAD, F), jnp.float32),
        ],
    )(Zp, Mp, Ap, Bp, emb96, wm2, wa2, wb2, Wm0)


def _g_body(dif_ref, wr_ref, g_ref):
    d = dif_ref[...]
    d2 = jnp.sum(d * d, axis=1, keepdims=True)
    dij = jnp.sqrt(d2 + 1e-12)
    mu = lax.broadcasted_iota(jnp.int32, (1, K), 1).astype(jnp.float32) * (CUTOFF / (K - 1))
    t = dij - mu
    rbf = jnp.exp(-4.0 * t * t)
    fc = 0.5 * (jnp.cos(jnp.pi * jnp.clip(dij / CUTOFF, 0.0, 1.0)) + 1.0)
    rbf = rbf * fc
    for b in range(NB):
        g_ref[b] = jnp.dot(rbf, wr_ref[b], preferred_element_type=jnp.float32)


def _g_call(d2c, W_rbf):
    return pl.pallas_call(
        _g_body,
        grid=(GE,),
        in_specs=[
            pl.BlockSpec((TE, F), lambda i: (i, 0)),
            pl.BlockSpec((NB, K, F), lambda i: (0, 0, 0)),
        ],
        out_specs=pl.BlockSpec((NB, TE, F), lambda i: (0, i, 0)),
        out_shape=jax.ShapeDtypeStruct((NB, E, F), jnp.float32),
    )(d2c, W_rbf)


def _upd_body(x_ref, P_ref, Wu_ref, bu_ref, Wm_ref, xo_ref, ho_ref):
    agg = P_ref[0] + P_ref[1]
    u = _ssp(jnp.dot(agg, Wu_ref[...], preferred_element_type=jnp.float32) + bu_ref[...])
    xn = x_ref[...] + u
    xo_ref[...] = xn
    ho_ref[...] = _ssp(jnp.dot(xn, Wm_ref[...], preferred_element_type=jnp.float32))


def _upd_call(x, P, Wu, bu, Wm):
    return pl.pallas_call(
        _upd_body,
        grid=(GN,),
        in_specs=[
            pl.BlockSpec((TN, F), lambda i: (i, 0)),
            pl.BlockSpec((2, TN, F), lambda i: (0, i, 0)),
            pl.BlockSpec((F, F), lambda i: (0, 0)),
            pl.BlockSpec((1, F), lambda i: (0, 0)),
            pl.BlockSpec((F, F), lambda i: (0, 0)),
        ],
        out_specs=[
            pl.BlockSpec((TN, F), lambda i: (i, 0)),
            pl.BlockSpec((TN, F), lambda i: (i, 0)),
        ],
        out_shape=[
            jax.ShapeDtypeStruct((NPAD, F), jnp.float32),
            jax.ShapeDtypeStruct((NPAD, F), jnp.float32),
        ],
    )(x, P, Wu, bu, Wm)


def _fin_body(x_ref, P_ref, Wu_ref, bu_ref, Wo_ref, bo_ref, mol_ref,
              vec_ref, acc_ref):
    i = pl.program_id(0)
    agg = P_ref[0] + P_ref[1]
    xn = x_ref[...] + _ssp(jnp.dot(agg, Wu_ref[...], preferred_element_type=jnp.float32) + bu_ref[...])
    out = jnp.dot(xn, Wo_ref[...], preferred_element_type=jnp.float32) + bo_ref[...]
    col = lax.broadcasted_iota(jnp.int32, (TN, F), 1)
    vec = jnp.where(col == 0, out, jnp.where(col == 1, jnp.maximum(out, 0.0), 0.0))
    vec_ref[...] = vec
    onehot = (lax.broadcasted_iota(jnp.int32, (NMOL, TN), 0) == mol_ref[0]).astype(jnp.float32)

    @pl.when(i == 0)
    def _():
        acc_ref[...] = jnp.zeros((NMOL, F), jnp.float32)

    acc_ref[...] += jnp.dot(onehot, vec, preferred_element_type=jnp.float32)


def _fin_call(x, P, Wu, bu, Wo, bo, mol2):
    return pl.pallas_call(
        _fin_body,
        grid=(GN,),
        in_specs=[
            pl.BlockSpec((TN, F), lambda i: (i, 0)),
            pl.BlockSpec((2, TN, F), lambda i: (0, i, 0)),
            pl.BlockSpec((F, F), lambda i: (0, 0)),
            pl.BlockSpec((1, F), lambda i: (0, 0)),
            pl.BlockSpec((F, F), lambda i: (0, 0)),
            pl.BlockSpec((1, F), lambda i: (0, 0)),
            pl.BlockSpec((1, 1, TN), lambda i: (i, 0, 0)),
        ],
        out_specs=[
            pl.BlockSpec((TN, F), lambda i: (i, 0)),
            pl.BlockSpec((NMOL, F), lambda i: (0, 0)),
        ],
        out_shape=[
            jax.ShapeDtypeStruct((NPAD, F), jnp.float32),
            jax.ShapeDtypeStruct((NMOL, F), jnp.float32),
        ],
    )(x, P, Wu, bu, Wo, bo, mol2)


def kernel(Z, R, M, QaAlpha, QaBeta, idx_i, idx_j, mol_idx,
           embed, w_m, w_a, w_b, W_rbf, W_msg, W_upd, b_upd, W_out, b_out):
    f32 = jnp.float32
    ii = idx_i.astype(jnp.int32)
    jj = idx_j.astype(jnp.int32)
    ii3 = ii.reshape(EC, CW)
    jj3 = jj.reshape(EC, CW)
    ij3 = jnp.stack([ii3, jj3], axis=1)

    W_rbf_p = W_rbf[:, :, COLPERM]
    W_msg_p = W_msg[:, :, COLPERM]
    W_upd_p = W_upd[:, COLPERM, :]

    R512 = jnp.zeros((N, F), f32).at[:, :3].set(R.astype(f32))
    dif = _d2_call(R512, ii3, jj3)
    G = _g_call(dif, W_rbf_p)

    Zp = jnp.zeros((NPAD, 1), jnp.int32).at[:N, 0].set(Z.astype(jnp.int32))
    Mp = jnp.zeros((NPAD, 1), f32).at[:N, 0].set(M)
    Ap = jnp.zeros((NPAD, 1), f32).at[:N, 0].set(QaAlpha)
    Bp = jnp.zeros((NPAD, 1), f32).at[:N, 0].set(QaBeta)
    emb96 = jnp.zeros((96, F), f32).at[:95].set(embed)
    wm2 = w_m.reshape(1, F)
    wa2 = w_a.reshape(1, F)
    wb2 = w_b.reshape(1, F)

    x, h = _init_call(Zp, Mp, Ap, Bp, emb96, wm2, wa2, wb2, W_msg_p[0])

    vec = acc = None
    for b in range(NB):
        P = _edge_calls[b](G, h, ij3)
        if b < NB - 1:
            x, h = _upd_call(x, P, W_upd_p[b], b_upd[b].reshape(1, F), W_msg_p[b + 1])
        else:
            Wo = jnp.zeros((F, F), f32).at[:, :2].set(W_out)
            bo = jnp.zeros((1, F), f32).at[0, :2].set(b_out)
            molp = (jnp.full((NPAD,), 1000, jnp.int32)
                    .at[:N].set(mol_idx.astype(jnp.int32)).reshape(GN, 1, TN))
            vec, acc = _fin_call(x, P, W_upd_p[b], b_upd[b].reshape(1, F), Wo, bo, molp)

    charges = acc[:, 0]
    Qa = vec[:N, 0]
    I_mol = acc[:, 1]
    return (charges, Qa, I_mol)
